# baseline (speedup 1.0000x reference)
import functools

import jax
import jax.numpy as jnp
from jax import lax
from jax.experimental import pallas as pl
from jax.experimental.pallas import tpu as pltpu
from jax.experimental.pallas import tpu_sc as plsc

_NC = 2
_NS = 16
_NW = _NC * _NS
_LANES = 16
_CHUNK = 128
_NBUF = 5


@functools.partial(jax.jit, static_argnames=("b", "n", "d"))
def _embed_sc(x3, tok_table, pos_table, b, n, d):
    total = b * n
    cpw = total // (_NW * _CHUNK)

    mesh = plsc.VectorSubcoreMesh(core_axis_name="c", subcore_axis_name="s")

    @functools.partial(
        pl.kernel,
        mesh=mesh,
        compiler_params=pltpu.CompilerParams(use_tc_tiling_on_sc=False),
        out_type=jax.ShapeDtypeStruct((total, d), jnp.float32),
        scratch_types=[
            pltpu.VMEM((cpw, _CHUNK), jnp.int32),
            pltpu.VMEM((_NBUF, _CHUNK, d), jnp.float32),
            pltpu.VMEM((2 * n, d), jnp.float32),
            pltpu.SemaphoreType.DMA((_NBUF,)),
            pltpu.SemaphoreType.DMA((_NBUF,)),
        ],
    )
    def k(x_hbm, tok_hbm, pos_hbm, out_hbm, idx_v, rows_v, pos_v, gsem, osem):
        wid = lax.axis_index("s") * _NC + lax.axis_index("c")
        pltpu.sync_copy(pos_hbm, pos_v.at[pl.ds(0, n)])
        pltpu.sync_copy(pos_hbm, pos_v.at[pl.ds(n, n)])
        pltpu.sync_copy(x_hbm.at[wid], idx_v)

        def fire_gather(i, b_):
            pltpu.make_async_copy(
                tok_hbm.at[idx_v.at[i]], rows_v.at[b_], gsem.at[b_]
            ).start()

        def out_slice(i):
            g = wid * cpw + i
            return out_hbm.at[pl.ds(g * _CHUNK, _CHUNK)]

        for b_ in range(_NBUF):
            fire_gather(b_, b_)

        @pl.loop(0, cpw, step=_NBUF)
        def _(i0):
            for b_ in range(_NBUF):
                i = i0 + b_
                pltpu.make_async_copy(
                    tok_hbm.at[idx_v.at[0]], rows_v.at[b_], gsem.at[b_]
                ).wait()
                g = wid * cpw + i
                po = (g * _CHUNK) % n

                @pl.loop(0, _CHUNK)
                def _(r):
                    for c in range(d // _LANES):
                        sl = pl.ds(c * _LANES, _LANES)
                        plsc.addupdate(rows_v.at[b_, r, sl], pos_v[po + r, sl])

                pltpu.make_async_copy(rows_v.at[b_], out_slice(i), osem.at[b_]).start()

                @pl.when(i + _NBUF < cpw)
                def _():
                    pltpu.make_async_copy(
                        rows_v.at[b_], out_slice(0), osem.at[b_]
                    ).wait()
                    fire_gather(i + _NBUF, b_)

        for b_ in range(_NBUF):
            pltpu.make_async_copy(rows_v.at[b_], out_slice(0), osem.at[b_]).wait()

    return k(x3, tok_table, pos_table)


def kernel(x, tok_table, pos_table):
    b, n = x.shape
    d = tok_table.shape[1]
    cpw = (b * n) // (_NW * _CHUNK)
    x3 = x.reshape(-1).astype(jnp.int32).reshape(_NW, cpw, _CHUNK)
    out = _embed_sc(x3, tok_table, pos_table, b, n, d)
    return out.reshape(b, n, d)
